# baseline (speedup 1.0000x reference)
import functools

import jax
import jax.numpy as jnp
from jax import lax
from jax.experimental import pallas as pl
from jax.experimental.pallas import tpu as pltpu
from jax.experimental.pallas import tpu_sc as plsc

N = 10000
E = 320000
D = 128

B = 128
NC = 2
NS = 16
NW = NC * NS
NB = 2560
PB = NB // NW
NPAD = 10240
RPT = NPAD // NS
ZB = RPT // 8

KD = 16
KS = 2
PBH = PB // 2

BLK = 400

_sc_mesh = plsc.VectorSubcoreMesh(core_axis_name="c", subcore_axis_name="s")


@functools.partial(
    pl.kernel,
    out_type=jax.ShapeDtypeStruct((NC, NPAD, D), jnp.float32),
    mesh=_sc_mesh,
    scratch_types=[
        pltpu.VMEM((PB, B), jnp.int32),
        pltpu.VMEM((B, D), jnp.float32),
        pltpu.VMEM((ZB, D), jnp.float32),
        pltpu.VMEM_SHARED((NPAD, D), jnp.float32),
        pltpu.SemaphoreType.DMA,
    ],
)
def _deg_kernel(dst_hbm, out_hbm, idx_v, ones_v, z_v, hist, dsem):
    c = lax.axis_index("c")
    s = lax.axis_index("s")
    w = c * NS + s
    onerow = jnp.ones((16,), jnp.float32)
    zrow = jnp.zeros((16,), jnp.float32)

    ld = pltpu.async_copy(dst_hbm.at[pl.ds(w * PB, PB)], idx_v, dsem)

    def initb(t, _):
        ones_v[t // 8, pl.ds(16 * (t % 8), 16)] = onerow
        return 0
    lax.fori_loop(0, B * 8, initb, 0)

    def initz(t, _):
        z_v[t // 8, pl.ds(16 * (t % 8), 16)] = zrow
        return 0
    lax.fori_loop(0, ZB * 8, initz, 0)

    def zcopy(k, _):
        pltpu.sync_copy(z_v, hist.at[pl.ds(s * RPT + k * ZB, ZB)])
        return 0
    lax.fori_loop(0, 8, zcopy, 0)
    ld.wait()
    plsc.subcore_barrier()

    def body(m, _):
        descs = [pltpu.async_copy(ones_v, hist.at[idx_v.at[m * KD + t]],
                                  dsem, add=True)
                 for t in range(KD)]
        for dsc in descs:
            dsc.wait()
        return 0
    lax.fori_loop(0, PB // KD, body, 0)
    plsc.subcore_barrier()

    pltpu.sync_copy(hist.at[pl.ds(s * RPT, RPT)],
                    out_hbm.at[c, pl.ds(s * RPT, RPT)])


@functools.partial(
    pl.kernel,
    out_type=jax.ShapeDtypeStruct((NC, NPAD, D), jnp.float32),
    mesh=_sc_mesh,
    scratch_types=[
        pltpu.VMEM((PBH, B), jnp.int32),
        pltpu.VMEM((PBH, B), jnp.int32),
        pltpu.VMEM((KS, B, D), jnp.float32),
        pltpu.VMEM_SHARED((NPAD, D), jnp.float32),
        pltpu.SemaphoreType.DMA,
        pltpu.SemaphoreType.DMA,
    ],
)
def _scatter_kernel(hp_hbm, src_hbm, dst_hbm, out_hbm,
                    src_v, dst_v, rows, acc, gsem, ssem):
    c = lax.axis_index("c")
    s = lax.axis_index("s")
    w = c * NS + s
    zrow = jnp.zeros((16,), jnp.float32)

    def initz(t, _):
        rows[0, t // 8, pl.ds(16 * (t % 8), 16)] = zrow
        return 0
    lax.fori_loop(0, B * 8, initz, 0)

    def zcopy(k, _):
        pltpu.sync_copy(rows.at[0], acc.at[pl.ds(s * RPT + k * B, B)])
        return 0
    lax.fori_loop(0, RPT // B, zcopy, 0)

    for h in range(2):
        pltpu.sync_copy(src_hbm.at[pl.ds(w * PB + h * PBH, PBH)], src_v)
        pltpu.sync_copy(dst_hbm.at[pl.ds(w * PB + h * PBH, PBH)], dst_v)

        pltpu.async_copy(hp_hbm.at[src_v.at[0]], rows.at[0], gsem)
        pltpu.async_copy(hp_hbm.at[src_v.at[1]], rows.at[1], gsem)

        if h == 0:
            plsc.subcore_barrier()

        def body(m, _):
            p = m % 2
            pltpu.make_async_copy(hp_hbm.at[src_v.at[m]], rows.at[p],
                                  gsem).wait()
            pltpu.async_copy(rows.at[p], acc.at[dst_v.at[m]], ssem, add=True)
            pltpu.make_async_copy(rows.at[p], acc.at[dst_v.at[m]],
                                  ssem).wait()
            pltpu.async_copy(hp_hbm.at[src_v.at[m + 2]], rows.at[p], gsem)
            return 0
        lax.fori_loop(0, PBH - 2, body, 0)

        def tail(m, _):
            p = m % 2
            pltpu.make_async_copy(hp_hbm.at[src_v.at[m]], rows.at[p],
                                  gsem).wait()
            pltpu.sync_copy(rows.at[p], acc.at[dst_v.at[m]], add=True)
            return 0
        lax.fori_loop(PBH - 2, PBH, tail, 0)
    plsc.subcore_barrier()

    pltpu.sync_copy(acc.at[pl.ds(s * RPT, RPT)],
                    out_hbm.at[c, pl.ds(s * RPT, RPT)])


def _row_spec(last):
    return pl.BlockSpec((BLK, last), lambda i: (i, 0))


def _full_spec(rows, cols):
    return pl.BlockSpec((rows, cols), lambda i: (0, 0))


def _tc1_body(x_ref, w1_ref, b1_ref, wc1_ref, dga_ref, dgb_ref,
              hp_ref, dis_ref):
    h0 = jnp.dot(x_ref[...], w1_ref[...], preferred_element_type=jnp.float32)
    h0 = jnp.maximum(h0 + b1_ref[...], 0.0)
    deg = dga_ref[...][:, :1] + dgb_ref[...][:, :1] + 1.0
    dis = lax.rsqrt(deg)
    hp_ref[...] = jnp.dot(h0, wc1_ref[...],
                          preferred_element_type=jnp.float32) * dis
    dis_ref[...] = jnp.broadcast_to(dis, (BLK, 16))


_tc1 = pl.pallas_call(
    _tc1_body,
    grid=(N // BLK,),
    in_specs=[_row_spec(D), _full_spec(D, D), _full_spec(1, D),
              _full_spec(D, D), _row_spec(D), _row_spec(D)],
    out_specs=[_row_spec(D), _row_spec(16)],
    out_shape=[jax.ShapeDtypeStruct((N, D), jnp.float32),
               jax.ShapeDtypeStruct((N, 16), jnp.float32)],
)


def _tc2_body(aa_ref, ab_ref, hp_ref, dis_ref, bc_ref, wc_ref, out_ref):
    dis = dis_ref[...][:, :1]
    h = dis * (aa_ref[...] + ab_ref[...] + hp_ref[...]) + bc_ref[...]
    out_ref[...] = jnp.dot(h, wc_ref[...],
                           preferred_element_type=jnp.float32) * dis


_tc2 = pl.pallas_call(
    _tc2_body,
    grid=(N // BLK,),
    in_specs=[_row_spec(D), _row_spec(D), _row_spec(D), _row_spec(16),
              _full_spec(1, D), _full_spec(D, D)],
    out_specs=[_row_spec(D)],
    out_shape=[jax.ShapeDtypeStruct((N, D), jnp.float32)],
)


def _tc3_body(aa_ref, ab_ref, hp_ref, dis_ref, bc_ref, w2_ref, b2_ref,
              out_ref):
    dis = dis_ref[...][:, :1]
    h = dis * (aa_ref[...] + ab_ref[...] + hp_ref[...]) + bc_ref[...]
    out_ref[...] = jnp.dot(h, w2_ref[...],
                           preferred_element_type=jnp.float32) + b2_ref[...]


_tc3 = pl.pallas_call(
    _tc3_body,
    grid=(N // BLK,),
    in_specs=[_row_spec(D), _row_spec(D), _row_spec(D), _row_spec(16),
              _full_spec(1, D), _full_spec(D, D), _full_spec(1, D)],
    out_specs=[_row_spec(D)],
    out_shape=[jax.ShapeDtypeStruct((N, D), jnp.float32)],
)


def kernel(x, edge_index, W1, b1, Wc1, bc1, Wc2, bc2, W2, b2):
    pad = NB * B - E
    filler = jnp.arange(pad, dtype=jnp.int32)
    src_flat = jnp.concatenate(
        [edge_index[0].astype(jnp.int32), filler % N])
    dst_flat = jnp.concatenate(
        [edge_index[1].astype(jnp.int32), N + filler % (NPAD - N)])
    src2d = src_flat.reshape(NB, B)
    dst2d = dst_flat.reshape(NB, B)

    deg2 = _deg_kernel(dst2d)
    hp1, dis = _tc1(x, W1, b1.reshape(1, D), Wc1, deg2[0], deg2[1])
    acc1 = _scatter_kernel(hp1, src2d, dst2d)
    (hp2,) = _tc2(acc1[0], acc1[1], hp1, dis, bc1.reshape(1, D), Wc2)
    acc2 = _scatter_kernel(hp2, src2d, dst2d)
    (out,) = _tc3(acc2[0], acc2[1], hp2, dis, bc2.reshape(1, D), W2,
                  b2.reshape(1, D))
    return out


# speedup vs baseline: 1.0050x; 1.0050x over previous
import functools

import jax
import jax.numpy as jnp
from jax import lax
from jax.experimental import pallas as pl
from jax.experimental.pallas import tpu as pltpu
from jax.experimental.pallas import tpu_sc as plsc

N = 10000
E = 320000
D = 128

B = 128
NC = 2
NS = 16
NW = NC * NS
NB = 2560
PB = NB // NW
NPAD = 10240
RPT = NPAD // NS
ZB = RPT // 8

KD = 16
KS = 2
PBH = PB // 2

BLK = 400

_sc_mesh = plsc.VectorSubcoreMesh(core_axis_name="c", subcore_axis_name="s")


@functools.partial(
    pl.kernel,
    out_type=jax.ShapeDtypeStruct((NC, NPAD, D), jnp.float32),
    mesh=_sc_mesh,
    scratch_types=[
        pltpu.VMEM((PB, B), jnp.int32),
        pltpu.VMEM((B, D), jnp.float32),
        pltpu.VMEM((ZB, D), jnp.float32),
        pltpu.VMEM_SHARED((NPAD, D), jnp.float32),
        pltpu.SemaphoreType.DMA,
    ],
)
def _deg_kernel(dst_hbm, out_hbm, idx_v, ones_v, z_v, hist, dsem):
    c = lax.axis_index("c")
    s = lax.axis_index("s")
    w = c * NS + s
    onerow = jnp.ones((16,), jnp.float32)
    zrow = jnp.zeros((16,), jnp.float32)

    ld = pltpu.async_copy(dst_hbm.at[pl.ds(w * PB, PB)], idx_v, dsem)

    def initb(t, _):
        ones_v[t // 8, pl.ds(16 * (t % 8), 16)] = onerow
        return 0
    lax.fori_loop(0, B * 8, initb, 0)

    def initz(t, _):
        z_v[t // 8, pl.ds(16 * (t % 8), 16)] = zrow
        return 0
    lax.fori_loop(0, ZB * 8, initz, 0)

    def zcopy(k, _):
        pltpu.sync_copy(z_v, hist.at[pl.ds(s * RPT + k * ZB, ZB)])
        return 0
    lax.fori_loop(0, 8, zcopy, 0)
    ld.wait()
    plsc.subcore_barrier()

    def body(m, _):
        descs = [pltpu.async_copy(ones_v, hist.at[idx_v.at[m * KD + t]],
                                  dsem, add=True)
                 for t in range(KD)]
        for dsc in descs:
            dsc.wait()
        return 0
    lax.fori_loop(0, PB // KD, body, 0)
    plsc.subcore_barrier()

    pltpu.sync_copy(hist.at[pl.ds(s * RPT, RPT)],
                    out_hbm.at[c, pl.ds(s * RPT, RPT)])


@functools.partial(
    pl.kernel,
    out_type=jax.ShapeDtypeStruct((NC, NPAD, D), jnp.float32),
    mesh=_sc_mesh,
    scratch_types=[
        pltpu.VMEM((PBH, B), jnp.int32),
        pltpu.VMEM((PBH, B), jnp.int32),
        pltpu.VMEM((KS, B, D), jnp.float32),
        pltpu.VMEM_SHARED((NPAD, D), jnp.float32),
        pltpu.SemaphoreType.DMA,
        pltpu.SemaphoreType.DMA,
    ],
)
def _scatter_kernel(hp_hbm, src_hbm, dst_hbm, out_hbm,
                    src_v, dst_v, rows, acc, gsem, ssem):
    c = lax.axis_index("c")
    s = lax.axis_index("s")
    w = c * NS + s
    zrow = jnp.zeros((16,), jnp.float32)

    def initz(t, _):
        rows[0, t // 8, pl.ds(16 * (t % 8), 16)] = zrow
        return 0
    lax.fori_loop(0, B * 8, initz, 0)

    def zcopy(k, _):
        pltpu.sync_copy(rows.at[0], acc.at[pl.ds(s * RPT + k * B, B)])
        return 0
    lax.fori_loop(0, RPT // B, zcopy, 0)

    for h in range(2):
        pltpu.sync_copy(src_hbm.at[pl.ds(w * PB + h * PBH, PBH)], src_v)
        pltpu.sync_copy(dst_hbm.at[pl.ds(w * PB + h * PBH, PBH)], dst_v)

        pltpu.async_copy(hp_hbm.at[src_v.at[0]], rows.at[0], gsem)
        pltpu.async_copy(hp_hbm.at[src_v.at[1]], rows.at[1], gsem)

        if h == 0:
            plsc.subcore_barrier()

        def body(m, _):
            p = m % 2
            pltpu.make_async_copy(hp_hbm.at[src_v.at[m]], rows.at[p],
                                  gsem).wait()
            pltpu.async_copy(rows.at[p], acc.at[dst_v.at[m]], ssem, add=True)
            pltpu.make_async_copy(rows.at[p], acc.at[dst_v.at[m]],
                                  ssem).wait()
            pltpu.async_copy(hp_hbm.at[src_v.at[m + 2]], rows.at[p], gsem)
            return 0
        lax.fori_loop(0, PBH - 2, body, 0)

        def tail(m, _):
            p = m % 2
            pltpu.make_async_copy(hp_hbm.at[src_v.at[m]], rows.at[p],
                                  gsem).wait()
            pltpu.sync_copy(rows.at[p], acc.at[dst_v.at[m]], add=True)
            return 0
        lax.fori_loop(PBH - 2, PBH, tail, 0)
    plsc.subcore_barrier()

    pltpu.sync_copy(acc.at[pl.ds(s * RPT, RPT)],
                    out_hbm.at[c, pl.ds(s * RPT, RPT)])


def _row_spec(last):
    return pl.BlockSpec((BLK, last), lambda i: (i, 0))


def _full_spec(rows, cols):
    return pl.BlockSpec((rows, cols), lambda i: (0, 0))


def _tc0_body(x_ref, w1_ref, b1_ref, wc1_ref, g_ref):
    h0 = jnp.dot(x_ref[...], w1_ref[...], preferred_element_type=jnp.float32)
    h0 = jnp.maximum(h0 + b1_ref[...], 0.0)
    g_ref[...] = jnp.dot(h0, wc1_ref[...], preferred_element_type=jnp.float32)


_tc0 = pl.pallas_call(
    _tc0_body,
    grid=(N // BLK,),
    in_specs=[_row_spec(D), _full_spec(D, D), _full_spec(1, D),
              _full_spec(D, D)],
    out_specs=[_row_spec(D)],
    out_shape=[jax.ShapeDtypeStruct((N, D), jnp.float32)],
)


def _tc1_body(g_ref, dga_ref, dgb_ref, hp_ref, dis_ref):
    deg = dga_ref[...][:, :1] + dgb_ref[...][:, :1] + 1.0
    dis = lax.rsqrt(deg)
    hp_ref[...] = g_ref[...] * dis
    dis_ref[...] = jnp.broadcast_to(dis, (BLK, 16))


_tc1 = pl.pallas_call(
    _tc1_body,
    grid=(N // BLK,),
    in_specs=[_row_spec(D), _row_spec(D), _row_spec(D)],
    out_specs=[_row_spec(D), _row_spec(16)],
    out_shape=[jax.ShapeDtypeStruct((N, D), jnp.float32),
               jax.ShapeDtypeStruct((N, 16), jnp.float32)],
)


def _tc2_body(aa_ref, ab_ref, hp_ref, dis_ref, bc_ref, wc_ref, out_ref):
    dis = dis_ref[...][:, :1]
    h = dis * (aa_ref[...] + ab_ref[...] + hp_ref[...]) + bc_ref[...]
    out_ref[...] = jnp.dot(h, wc_ref[...],
                           preferred_element_type=jnp.float32) * dis


_tc2 = pl.pallas_call(
    _tc2_body,
    grid=(N // BLK,),
    in_specs=[_row_spec(D), _row_spec(D), _row_spec(D), _row_spec(16),
              _full_spec(1, D), _full_spec(D, D)],
    out_specs=[_row_spec(D)],
    out_shape=[jax.ShapeDtypeStruct((N, D), jnp.float32)],
)


def _tc3_body(aa_ref, ab_ref, hp_ref, dis_ref, bc_ref, w2_ref, b2_ref,
              out_ref):
    dis = dis_ref[...][:, :1]
    h = dis * (aa_ref[...] + ab_ref[...] + hp_ref[...]) + bc_ref[...]
    out_ref[...] = jnp.dot(h, w2_ref[...],
                           preferred_element_type=jnp.float32) + b2_ref[...]


_tc3 = pl.pallas_call(
    _tc3_body,
    grid=(N // BLK,),
    in_specs=[_row_spec(D), _row_spec(D), _row_spec(D), _row_spec(16),
              _full_spec(1, D), _full_spec(D, D), _full_spec(1, D)],
    out_specs=[_row_spec(D)],
    out_shape=[jax.ShapeDtypeStruct((N, D), jnp.float32)],
)


def kernel(x, edge_index, W1, b1, Wc1, bc1, Wc2, bc2, W2, b2):
    pad = NB * B - E
    filler = jnp.arange(pad, dtype=jnp.int32)
    src_flat = jnp.concatenate(
        [edge_index[0].astype(jnp.int32), filler % N])
    dst_flat = jnp.concatenate(
        [edge_index[1].astype(jnp.int32), N + filler % (NPAD - N)])
    src2d = src_flat.reshape(NB, B)
    dst2d = dst_flat.reshape(NB, B)

    deg2 = _deg_kernel(dst2d)
    (g1,) = _tc0(x, W1, b1.reshape(1, D), Wc1)
    hp1, dis = _tc1(g1, deg2[0], deg2[1])
    acc1 = _scatter_kernel(hp1, src2d, dst2d)
    (hp2,) = _tc2(acc1[0], acc1[1], hp1, dis, bc1.reshape(1, D), Wc2)
    acc2 = _scatter_kernel(hp2, src2d, dst2d)
    (out,) = _tc3(acc2[0], acc2[1], hp2, dis, bc2.reshape(1, D), W2,
                  b2.reshape(1, D))
    return out
